# baseline (speedup 1.0000x reference)
import jax
import jax.numpy as jnp
from jax.experimental import pallas as pl
from jax.experimental.pallas import tpu as pltpu

_NEG_SLOPE = 0.2
_CHUNK = 256


def _gat_dense_kernel(x_ref, adj_hbm, W_ref, a_ref, out_ref, adj_vmem, sems):
    N = x_ref.shape[0]
    d = W_ref.shape[1]
    n_chunks = N // _CHUNK

    def chunk_copy(k):
        return pltpu.make_async_copy(
            adj_hbm.at[pl.ds(k * _CHUNK, _CHUNK), :],
            adj_vmem.at[k % 2],
            sems.at[k % 2],
        )

    chunk_copy(0).start()
    h = jnp.dot(x_ref[...], W_ref[...], preferred_element_type=jnp.float32)
    ns = jnp.dot(h, -a_ref[0, :d])
    nt = jnp.dot(h, -a_ref[0, d:])
    nt_row = nt[None, :]
    for k in range(n_chunks):
        if k + 1 < n_chunks:
            chunk_copy(k + 1).start()
        chunk_copy(k).wait()
        u = ns[k * _CHUNK:(k + 1) * _CHUNK, None] + nt_row
        arg = jnp.minimum(u, _NEG_SLOPE * u)
        e = jnp.where(adj_vmem[k % 2] != 0, jnp.exp(arg), 0.0)
        rowsum = jnp.sum(e, axis=1, keepdims=True)
        num = jnp.dot(e, h, preferred_element_type=jnp.float32)
        hp = num / rowsum
        hp = jnp.where(jnp.isnan(hp), 0.0, hp)
        out_ref[pl.ds(k * _CHUNK, _CHUNK), :] = jnp.where(
            hp > 0, hp, jnp.exp(jnp.minimum(hp, 0.0)) - 1.0
        )


def kernel(input, adj, W, a):
    B, N, d_in = input.shape
    d_out = W.shape[1]
    x2 = input.reshape(B * N, d_in)
    adj2 = adj.reshape(B * N, N)
    out = pl.pallas_call(
        _gat_dense_kernel,
        in_specs=[
            pl.BlockSpec(memory_space=pltpu.MemorySpace.VMEM),
            pl.BlockSpec(memory_space=pltpu.MemorySpace.HBM),
            pl.BlockSpec(memory_space=pltpu.MemorySpace.VMEM),
            pl.BlockSpec(memory_space=pltpu.MemorySpace.VMEM),
        ],
        out_specs=pl.BlockSpec(memory_space=pltpu.MemorySpace.VMEM),
        out_shape=jax.ShapeDtypeStruct((B * N, d_out), jnp.float32),
        scratch_shapes=[
            pltpu.VMEM((2, _CHUNK, B * N), jnp.int32),
            pltpu.SemaphoreType.DMA((2,)),
        ],
    )(x2, adj2, W, a)
    return out.reshape(B, N, d_out)


# speedup vs baseline: 1.0862x; 1.0862x over previous
import jax
import jax.numpy as jnp
from jax.experimental import pallas as pl
from jax.experimental.pallas import tpu as pltpu

_NEG_SLOPE = 0.2
_CHUNK = 512


def _gat_dense_kernel(x_ref, adj_hbm, W_ref, a_ref, out_ref, adj_vmem, sems):
    N = x_ref.shape[0]
    d = W_ref.shape[1]
    n_chunks = N // _CHUNK

    def chunk_copy(k):
        return pltpu.make_async_copy(
            adj_hbm.at[pl.ds(k * _CHUNK, _CHUNK), :],
            adj_vmem.at[k],
            sems.at[k],
        )

    for k in range(n_chunks):
        chunk_copy(k).start()
    h = jnp.dot(x_ref[...], W_ref[...], preferred_element_type=jnp.float32)
    ns = jnp.dot(h, -a_ref[0, :d])
    nt = jnp.dot(h, -a_ref[0, d:])
    nt_row = nt[None, :]
    for k in range(n_chunks):
        chunk_copy(k).wait()
        u = ns[k * _CHUNK:(k + 1) * _CHUNK, None] + nt_row
        arg = jnp.minimum(u, _NEG_SLOPE * u)
        e = jnp.where(adj_vmem[k] != 0, jnp.exp(arg), 0.0)
        rowsum = jnp.sum(e, axis=1, keepdims=True)
        num = jnp.dot(e, h, preferred_element_type=jnp.float32)
        hp = num / rowsum
        hp = jnp.where(jnp.isnan(hp), 0.0, hp)
        out_ref[pl.ds(k * _CHUNK, _CHUNK), :] = jnp.where(
            hp > 0, hp, jnp.exp(jnp.minimum(hp, 0.0)) - 1.0
        )


def kernel(input, adj, W, a):
    B, N, d_in = input.shape
    d_out = W.shape[1]
    x2 = input.reshape(B * N, d_in)
    adj2 = adj.reshape(B * N, N)
    n_chunks = (B * N) // _CHUNK
    out = pl.pallas_call(
        _gat_dense_kernel,
        in_specs=[
            pl.BlockSpec(memory_space=pltpu.MemorySpace.VMEM),
            pl.BlockSpec(memory_space=pltpu.MemorySpace.HBM),
            pl.BlockSpec(memory_space=pltpu.MemorySpace.VMEM),
            pl.BlockSpec(memory_space=pltpu.MemorySpace.VMEM),
        ],
        out_specs=pl.BlockSpec(memory_space=pltpu.MemorySpace.VMEM),
        out_shape=jax.ShapeDtypeStruct((B * N, d_out), jnp.float32),
        scratch_shapes=[
            pltpu.VMEM((n_chunks, _CHUNK, B * N), jnp.int32),
            pltpu.SemaphoreType.DMA((n_chunks,)),
        ],
    )(x2, adj2, W, a)
    return out.reshape(B, N, d_out)


# speedup vs baseline: 1.0992x; 1.0120x over previous
import jax
import jax.numpy as jnp
from jax.experimental import pallas as pl

_NEG_SLOPE = 0.2


def _gat_dense_kernel(x_ref, adj_ref, W_ref, a_ref, out_ref):
    h = jnp.dot(x_ref[...], W_ref[...], preferred_element_type=jnp.float32)
    d = W_ref.shape[1]
    a_src = a_ref[0, :d]
    a_dst = a_ref[0, d:]
    ns = jnp.dot(h, -a_src)
    nt = jnp.dot(h, -a_dst)
    u = ns[:, None] + nt[None, :]
    arg = jnp.minimum(u, _NEG_SLOPE * u)
    e = jnp.where(adj_ref[...] != 0, jnp.exp(arg), 0.0)
    rowsum = jnp.sum(e, axis=1, keepdims=True)
    num = jnp.dot(e, h, preferred_element_type=jnp.float32)
    hp = num / rowsum
    hp = jnp.where(jnp.isnan(hp), 0.0, hp)
    out_ref[...] = jnp.where(hp > 0, hp, jnp.exp(jnp.minimum(hp, 0.0)) - 1.0)


def kernel(input, adj, W, a):
    B, N, d_in = input.shape
    d_out = W.shape[1]
    x2 = input.reshape(B * N, d_in)
    adj2 = adj.reshape(B * N, N)
    out = pl.pallas_call(
        _gat_dense_kernel,
        out_shape=jax.ShapeDtypeStruct((B * N, d_out), jnp.float32),
    )(x2, adj2, W, a)
    return out.reshape(B, N, d_out)
